# baseline (speedup 1.0000x reference)
import functools

import jax
import jax.numpy as jnp
from jax.experimental import pallas as pl
from jax.experimental.pallas import tpu as pltpu

_VMEM_LIMIT = 60 * 1024 * 1024
_TILE_BYTES = 4 * 1024 * 1024
_TILE_W = 16384
_NCORES = 2


def _mlp_att(p, w1t, b1, w2, b2):
    h = jnp.sum(w1t * p, axis=0, keepdims=True) + b1
    h = jnp.maximum(h, 0.0)
    a = jnp.sum(w2 * h, axis=1, keepdims=True) + b2
    return jax.nn.sigmoid(a)


def _ca_kernel_pipe(x_ref, w1t_ref, b1_ref, w2_ref, b2_ref, o_ref,
                    slab_ref, acc_ref, att0_ref, att1_ref,
                    *, kt, th, nb, inv_hw):
    j = pl.program_id(1)
    k = pl.program_id(2)
    par = jax.lax.rem(j, 2)

    @pl.when(j >= 1)
    def _write():
        @pl.when(par == 1)
        def _w0():
            att = att0_ref[...][:, :, None]
            if o_ref.dtype != jnp.float32:
                att = att.astype(o_ref.dtype)
            blk = slab_ref[:, pl.ds(k * th, th), :]
            o_ref[0] = (blk * att).astype(o_ref.dtype)

        @pl.when(par == 0)
        def _w1():
            att = att1_ref[...][:, :, None]
            if o_ref.dtype != jnp.float32:
                att = att.astype(o_ref.dtype)
            blk = slab_ref[:, pl.ds(k * th, th), :]
            o_ref[0] = (blk * att).astype(o_ref.dtype)

    @pl.when(j < nb)
    def _read():
        x_blk = x_ref[0]

        @pl.when(k == 0)
        def _init():
            acc_ref[...] = jnp.zeros_like(acc_ref)

        slab_ref[:, pl.ds(k * th, th), :] = x_blk
        acc_ref[...] += jnp.sum(x_blk.astype(jnp.float32), axis=1)

        @pl.when(k == kt - 1)
        def _finalize():
            p = jnp.sum(acc_ref[...], axis=-1, keepdims=True) * inv_hw
            a = _mlp_att(p, w1t_ref[...], b1_ref[...],
                         w2_ref[...], b2_ref[...])

            @pl.when(par == 0)
            def _a0():
                att0_ref[...] = a

            @pl.when(par == 1)
            def _a1():
                att1_ref[...] = a


def _ca_layer_pipe(x, w1t, b1, w2, b2, th):
    B, C, H, W = x.shape
    cr = w1t.shape[1]
    inv_hw = 1.0 / (H * W)
    kt = H // th
    nb = B // _NCORES

    def x_map(c, j, k):
        b_r = c * nb + jnp.minimum(j, nb - 1)
        k_r = jnp.where(j == nb, kt - 1, k)
        return (b_r, 0, k_r, 0)

    def o_map(c, j, k):
        b_w = c * nb + jnp.maximum(j - 1, 0)
        k_w = jnp.where(j == 0, 0, k)
        return (b_w, 0, k_w, 0)

    mlp_specs = [pl.BlockSpec((C, cr), lambda c, j, k: (0, 0)),
                 pl.BlockSpec((1, cr), lambda c, j, k: (0, 0)),
                 pl.BlockSpec((C, cr), lambda c, j, k: (0, 0)),
                 pl.BlockSpec((C, 1), lambda c, j, k: (0, 0))]

    return pl.pallas_call(
        functools.partial(_ca_kernel_pipe, kt=kt, th=th, nb=nb,
                          inv_hw=inv_hw),
        out_shape=jax.ShapeDtypeStruct((B, C, H, W), x.dtype),
        grid=(_NCORES, nb + 1, kt),
        in_specs=[pl.BlockSpec((1, C, th, W), x_map)] + mlp_specs,
        out_specs=pl.BlockSpec((1, C, th, W), o_map),
        scratch_shapes=[pltpu.VMEM((C, H, W), x.dtype),
                        pltpu.VMEM((C, W), jnp.float32),
                        pltpu.VMEM((C, 1), jnp.float32),
                        pltpu.VMEM((C, 1), jnp.float32)],
        compiler_params=pltpu.CompilerParams(
            dimension_semantics=("parallel", "arbitrary", "arbitrary"),
            vmem_limit_bytes=_VMEM_LIMIT),
    )(x, w1t, b1, w2, b2)


def _ca_kernel4d(x_ref, w1t_ref, b1_ref, w2_ref, b2_ref, o_ref,
                 slab_ref, acc_ref, att_ref, *, kt, th, inv_hw):
    k = pl.program_id(1)

    @pl.when(k == 0)
    def _init():
        acc_ref[...] = jnp.zeros_like(acc_ref)

    @pl.when(k < kt)
    def _read_phase():
        x_blk = x_ref[0]
        slab_ref[:, pl.ds(k * th, th), :] = x_blk
        acc_ref[...] += jnp.sum(x_blk.astype(jnp.float32), axis=1)

    @pl.when(k == kt - 1)
    def _finalize():
        p = jnp.sum(acc_ref[...], axis=-1, keepdims=True) * inv_hw
        att_ref[...] = _mlp_att(p, w1t_ref[...], b1_ref[...],
                                w2_ref[...], b2_ref[...])

    @pl.when(k >= kt)
    def _write_phase():
        t = k - kt
        att = att_ref[...][:, :, None]
        if o_ref.dtype != jnp.float32:
            att = att.astype(o_ref.dtype)
        blk = slab_ref[:, pl.ds(t * th, th), :]
        o_ref[0] = (blk * att).astype(o_ref.dtype)


def _ca_layer4d(x, w1t, b1, w2, b2, th):
    B, C, H, W = x.shape
    cr = w1t.shape[1]
    inv_hw = 1.0 / (H * W)
    kt = H // th

    mlp_specs = [pl.BlockSpec((C, cr), lambda b, k: (0, 0)),
                 pl.BlockSpec((1, cr), lambda b, k: (0, 0)),
                 pl.BlockSpec((C, cr), lambda b, k: (0, 0)),
                 pl.BlockSpec((C, 1), lambda b, k: (0, 0))]

    return pl.pallas_call(
        functools.partial(_ca_kernel4d, kt=kt, th=th, inv_hw=inv_hw),
        out_shape=jax.ShapeDtypeStruct((B, C, H, W), x.dtype),
        grid=(B, 2 * kt),
        in_specs=[pl.BlockSpec((1, C, th, W),
                               lambda b, k: (b, 0, jnp.minimum(k, kt - 1), 0))]
                 + mlp_specs,
        out_specs=pl.BlockSpec((1, C, th, W),
                               lambda b, k: (b, 0, jnp.maximum(k - kt, 0), 0)),
        scratch_shapes=[pltpu.VMEM((C, H, W), x.dtype),
                        pltpu.VMEM((C, W), jnp.float32),
                        pltpu.VMEM((C, 1), jnp.float32)],
        compiler_params=pltpu.CompilerParams(
            dimension_semantics=("parallel", "arbitrary"),
            vmem_limit_bytes=_VMEM_LIMIT),
    )(x, w1t, b1, w2, b2)


def _tree_sum128(x_blk):
    tw = x_blk.shape[-1]
    n = max(tw // 128, 1)
    chunks = [x_blk[:, i * 128:(i + 1) * 128].astype(jnp.float32)
              for i in range(n)]
    while len(chunks) > 1:
        nxt = [chunks[i] + chunks[i + 1] for i in range(0, len(chunks) - 1, 2)]
        if len(chunks) % 2:
            nxt.append(chunks[-1])
        chunks = nxt
    return chunks[0]


def _ca_kernel_flat(x_ref, w1t_ref, b1_ref, w2_ref, b2_ref, o_ref,
                    slab_ref, acc_ref, att_ref, *, kt, tw, last_w, inv_hw):
    k = pl.program_id(1)

    @pl.when(k == 0)
    def _init():
        acc_ref[...] = jnp.zeros_like(acc_ref)

    @pl.when(k < kt)
    def _read_phase():
        x_blk = x_ref[0]
        slab_ref[:, pl.ds(k * tw, tw)] = x_blk
        if last_w == tw:
            acc_ref[...] += _tree_sum128(x_blk)
        else:
            @pl.when(k < kt - 1)
            def _full():
                acc_ref[...] += _tree_sum128(x_blk)

            @pl.when(k == kt - 1)
            def _masked():
                lane = jax.lax.broadcasted_iota(jnp.int32, x_blk.shape, 1)
                masked = jnp.where(lane < last_w, x_blk,
                                   jnp.zeros_like(x_blk))
                acc_ref[...] += _tree_sum128(masked)

    @pl.when(k == kt - 1)
    def _finalize():
        p = jnp.sum(acc_ref[...], axis=-1, keepdims=True) * inv_hw
        att_ref[...] = _mlp_att(p, w1t_ref[...], b1_ref[...],
                                w2_ref[...], b2_ref[...])

    @pl.when(k >= kt)
    def _write_phase():
        t = k - kt
        att = att_ref[...]
        if o_ref.dtype != jnp.float32:
            att = att.astype(o_ref.dtype)
        blk = slab_ref[:, pl.ds(t * tw, tw)]
        o_ref[0] = (blk * att).astype(o_ref.dtype)


def _ca_layer_flat(x, w1t, b1, w2, b2):
    B, C, H, W = x.shape
    cr = w1t.shape[1]
    hw = H * W
    inv_hw = 1.0 / hw
    x_flat = x.reshape(B, C, hw)

    if hw % 128 == 0 or hw < 128:
        tw = min(_TILE_W, hw)
    else:
        tw = min(_TILE_W, (hw // 128) * 128)
    kt = -(-hw // tw)
    last_w = hw - (kt - 1) * tw

    mlp_specs = [pl.BlockSpec((C, cr), lambda b, k: (0, 0)),
                 pl.BlockSpec((1, cr), lambda b, k: (0, 0)),
                 pl.BlockSpec((C, cr), lambda b, k: (0, 0)),
                 pl.BlockSpec((C, 1), lambda b, k: (0, 0))]

    y = pl.pallas_call(
        functools.partial(_ca_kernel_flat, kt=kt, tw=tw, last_w=last_w,
                          inv_hw=inv_hw),
        out_shape=jax.ShapeDtypeStruct((B, C, hw), x.dtype),
        grid=(B, 2 * kt),
        in_specs=[pl.BlockSpec((1, C, tw),
                               lambda b, k: (b, 0, jnp.minimum(k, kt - 1)))]
                 + mlp_specs,
        out_specs=pl.BlockSpec((1, C, tw),
                               lambda b, k: (b, 0, jnp.maximum(k - kt, 0))),
        scratch_shapes=[pltpu.VMEM((C, kt * tw), x.dtype),
                        pltpu.VMEM((C, 128), jnp.float32),
                        pltpu.VMEM((C, 1), jnp.float32)],
        compiler_params=pltpu.CompilerParams(
            dimension_semantics=("parallel", "arbitrary"),
            vmem_limit_bytes=_VMEM_LIMIT),
    )(x_flat, w1t, b1, w2, b2)
    return y.reshape(B, C, H, W)


def _pick_th(C, H, W, itemsize, tile_bytes=_TILE_BYTES):
    if W % 128 != 0 or H % 8 != 0:
        return None
    best = None
    for th in range(8, H + 1, 8):
        if H % th == 0 and C * th * W * itemsize <= tile_bytes:
            best = th
    return best if best is not None else 8


def kernel(x, conv1_w, conv1_b, conv2_w, conv2_b):
    B, C, H, W = x.shape
    cr = conv1_w.shape[0]

    w1t = conv1_w.astype(jnp.float32).T
    b1 = conv1_b.astype(jnp.float32).reshape(1, cr)
    w2 = conv2_w.astype(jnp.float32)
    b2 = conv2_b.astype(jnp.float32).reshape(C, 1)

    th = _pick_th(C, H, W, x.dtype.itemsize)
    if th is not None:
        if B % _NCORES == 0 and B >= 2 * _NCORES:
            slab_bytes = C * H * W * x.dtype.itemsize
            tile_budget = max((32 * 1024 * 1024 - slab_bytes) // 4,
                              _TILE_BYTES)
            th_pipe = _pick_th(C, H, W, x.dtype.itemsize, tile_budget)
            return _ca_layer_pipe(x, w1t, b1, w2, b2, th_pipe)
        return _ca_layer4d(x, w1t, b1, w2, b2, th)
    return _ca_layer_flat(x, w1t, b1, w2, b2)


# speedup vs baseline: 1.0322x; 1.0322x over previous
import functools

import jax
import jax.numpy as jnp
from jax.experimental import pallas as pl
from jax.experimental.pallas import tpu as pltpu

_VMEM_LIMIT = 60 * 1024 * 1024
_TILE_BYTES = 4 * 1024 * 1024
_TILE_W = 16384
_NCORES = 2


def _mlp_att(p, w1t, b1, w2, b2):
    h = jnp.sum(w1t * p, axis=0, keepdims=True) + b1
    h = jnp.maximum(h, 0.0)
    a = jnp.sum(w2 * h, axis=1, keepdims=True) + b2
    return jax.nn.sigmoid(a)


def _ca_kernel_pipe(x_ref, w1t_ref, b1_ref, w2_ref, b2_ref, o_ref,
                    slab_ref, acc_ref, att0_ref, att1_ref,
                    *, kt, th, nb, inv_hw):
    j = pl.program_id(1)
    k = pl.program_id(2)
    par = jax.lax.rem(j, 2)

    @pl.when(j >= 1)
    def _write():
        @pl.when(par == 1)
        def _w0():
            att = att0_ref[...][:, :, None]
            if o_ref.dtype != jnp.float32:
                att = att.astype(o_ref.dtype)
            blk = slab_ref[:, pl.ds(k * th, th), :]
            o_ref[0] = (blk * att).astype(o_ref.dtype)

        @pl.when(par == 0)
        def _w1():
            att = att1_ref[...][:, :, None]
            if o_ref.dtype != jnp.float32:
                att = att.astype(o_ref.dtype)
            blk = slab_ref[:, pl.ds(k * th, th), :]
            o_ref[0] = (blk * att).astype(o_ref.dtype)

    @pl.when(j < nb)
    def _read():
        x_blk = x_ref[0]

        @pl.when(k == 0)
        def _init():
            acc_ref[...] = jnp.zeros_like(acc_ref)

        slab_ref[:, pl.ds(k * th, th), :] = x_blk
        acc_ref[...] += jnp.sum(x_blk.astype(jnp.float32), axis=1)

        @pl.when(k == kt - 1)
        def _finalize():
            p = jnp.sum(acc_ref[...], axis=-1, keepdims=True) * inv_hw
            a = _mlp_att(p, w1t_ref[...], b1_ref[...],
                         w2_ref[...], b2_ref[...])

            @pl.when(par == 0)
            def _a0():
                att0_ref[...] = a

            @pl.when(par == 1)
            def _a1():
                att1_ref[...] = a


def _ca_layer_pipe(x, w1t, b1, w2, b2, th):
    B, C, H, W = x.shape
    cr = w1t.shape[1]
    inv_hw = 1.0 / (H * W)
    kt = H // th
    nb = B // _NCORES

    def x_map(c, j, k):
        b_r = c * nb + jnp.minimum(j, nb - 1)
        k_r = jnp.where(j == nb, kt - 1, k)
        return (b_r, 0, k_r, 0)

    def o_map(c, j, k):
        b_w = c * nb + jnp.maximum(j - 1, 0)
        k_w = jnp.where(j == 0, 0, k)
        return (b_w, 0, k_w, 0)

    mlp_specs = [pl.BlockSpec((C, cr), lambda c, j, k: (0, 0)),
                 pl.BlockSpec((1, cr), lambda c, j, k: (0, 0)),
                 pl.BlockSpec((C, cr), lambda c, j, k: (0, 0)),
                 pl.BlockSpec((C, 1), lambda c, j, k: (0, 0))]

    return pl.pallas_call(
        functools.partial(_ca_kernel_pipe, kt=kt, th=th, nb=nb,
                          inv_hw=inv_hw),
        out_shape=jax.ShapeDtypeStruct((B, C, H, W), x.dtype),
        grid=(_NCORES, nb + 1, kt),
        in_specs=[pl.BlockSpec((1, C, th, W), x_map)] + mlp_specs,
        out_specs=pl.BlockSpec((1, C, th, W), o_map),
        scratch_shapes=[pltpu.VMEM((C, H, W), x.dtype),
                        pltpu.VMEM((C, W), jnp.float32),
                        pltpu.VMEM((C, 1), jnp.float32),
                        pltpu.VMEM((C, 1), jnp.float32)],
        compiler_params=pltpu.CompilerParams(
            dimension_semantics=("parallel", "arbitrary", "arbitrary"),
            vmem_limit_bytes=_VMEM_LIMIT),
    )(x, w1t, b1, w2, b2)


def _ca_kernel4d(x_ref, w1t_ref, b1_ref, w2_ref, b2_ref, o_ref,
                 slab_ref, acc_ref, att_ref, *, kt, th, inv_hw):
    k = pl.program_id(1)

    @pl.when(k == 0)
    def _init():
        acc_ref[...] = jnp.zeros_like(acc_ref)

    @pl.when(k < kt)
    def _read_phase():
        x_blk = x_ref[0]
        slab_ref[:, pl.ds(k * th, th), :] = x_blk
        acc_ref[...] += jnp.sum(x_blk.astype(jnp.float32), axis=1)

    @pl.when(k == kt - 1)
    def _finalize():
        p = jnp.sum(acc_ref[...], axis=-1, keepdims=True) * inv_hw
        att_ref[...] = _mlp_att(p, w1t_ref[...], b1_ref[...],
                                w2_ref[...], b2_ref[...])

    @pl.when(k >= kt)
    def _write_phase():
        t = k - kt
        att = att_ref[...][:, :, None]
        if o_ref.dtype != jnp.float32:
            att = att.astype(o_ref.dtype)
        blk = slab_ref[:, pl.ds(t * th, th), :]
        o_ref[0] = (blk * att).astype(o_ref.dtype)


def _ca_layer4d(x, w1t, b1, w2, b2, th):
    B, C, H, W = x.shape
    cr = w1t.shape[1]
    inv_hw = 1.0 / (H * W)
    kt = H // th

    mlp_specs = [pl.BlockSpec((C, cr), lambda b, k: (0, 0)),
                 pl.BlockSpec((1, cr), lambda b, k: (0, 0)),
                 pl.BlockSpec((C, cr), lambda b, k: (0, 0)),
                 pl.BlockSpec((C, 1), lambda b, k: (0, 0))]

    return pl.pallas_call(
        functools.partial(_ca_kernel4d, kt=kt, th=th, inv_hw=inv_hw),
        out_shape=jax.ShapeDtypeStruct((B, C, H, W), x.dtype),
        grid=(B, 2 * kt),
        in_specs=[pl.BlockSpec((1, C, th, W),
                               lambda b, k: (b, 0, jnp.minimum(k, kt - 1), 0))]
                 + mlp_specs,
        out_specs=pl.BlockSpec((1, C, th, W),
                               lambda b, k: (b, 0, jnp.maximum(k - kt, 0), 0)),
        scratch_shapes=[pltpu.VMEM((C, H, W), x.dtype),
                        pltpu.VMEM((C, W), jnp.float32),
                        pltpu.VMEM((C, 1), jnp.float32)],
        compiler_params=pltpu.CompilerParams(
            dimension_semantics=("parallel", "arbitrary"),
            vmem_limit_bytes=_VMEM_LIMIT),
    )(x, w1t, b1, w2, b2)


def _tree_sum128(x_blk):
    tw = x_blk.shape[-1]
    n = max(tw // 128, 1)
    chunks = [x_blk[:, i * 128:(i + 1) * 128].astype(jnp.float32)
              for i in range(n)]
    while len(chunks) > 1:
        nxt = [chunks[i] + chunks[i + 1] for i in range(0, len(chunks) - 1, 2)]
        if len(chunks) % 2:
            nxt.append(chunks[-1])
        chunks = nxt
    return chunks[0]


def _ca_kernel_flat(x_ref, w1t_ref, b1_ref, w2_ref, b2_ref, o_ref,
                    slab_ref, acc_ref, att_ref, *, kt, tw, last_w, inv_hw):
    k = pl.program_id(1)

    @pl.when(k == 0)
    def _init():
        acc_ref[...] = jnp.zeros_like(acc_ref)

    @pl.when(k < kt)
    def _read_phase():
        x_blk = x_ref[0]
        slab_ref[:, pl.ds(k * tw, tw)] = x_blk
        if last_w == tw:
            acc_ref[...] += _tree_sum128(x_blk)
        else:
            @pl.when(k < kt - 1)
            def _full():
                acc_ref[...] += _tree_sum128(x_blk)

            @pl.when(k == kt - 1)
            def _masked():
                lane = jax.lax.broadcasted_iota(jnp.int32, x_blk.shape, 1)
                masked = jnp.where(lane < last_w, x_blk,
                                   jnp.zeros_like(x_blk))
                acc_ref[...] += _tree_sum128(masked)

    @pl.when(k == kt - 1)
    def _finalize():
        p = jnp.sum(acc_ref[...], axis=-1, keepdims=True) * inv_hw
        att_ref[...] = _mlp_att(p, w1t_ref[...], b1_ref[...],
                                w2_ref[...], b2_ref[...])

    @pl.when(k >= kt)
    def _write_phase():
        t = k - kt
        att = att_ref[...]
        if o_ref.dtype != jnp.float32:
            att = att.astype(o_ref.dtype)
        blk = slab_ref[:, pl.ds(t * tw, tw)]
        o_ref[0] = (blk * att).astype(o_ref.dtype)


def _ca_layer_flat(x, w1t, b1, w2, b2):
    B, C, H, W = x.shape
    cr = w1t.shape[1]
    hw = H * W
    inv_hw = 1.0 / hw
    x_flat = x.reshape(B, C, hw)

    if hw % 128 == 0 or hw < 128:
        tw = min(_TILE_W, hw)
    else:
        tw = min(_TILE_W, (hw // 128) * 128)
    kt = -(-hw // tw)
    last_w = hw - (kt - 1) * tw

    mlp_specs = [pl.BlockSpec((C, cr), lambda b, k: (0, 0)),
                 pl.BlockSpec((1, cr), lambda b, k: (0, 0)),
                 pl.BlockSpec((C, cr), lambda b, k: (0, 0)),
                 pl.BlockSpec((C, 1), lambda b, k: (0, 0))]

    y = pl.pallas_call(
        functools.partial(_ca_kernel_flat, kt=kt, tw=tw, last_w=last_w,
                          inv_hw=inv_hw),
        out_shape=jax.ShapeDtypeStruct((B, C, hw), x.dtype),
        grid=(B, 2 * kt),
        in_specs=[pl.BlockSpec((1, C, tw),
                               lambda b, k: (b, 0, jnp.minimum(k, kt - 1)))]
                 + mlp_specs,
        out_specs=pl.BlockSpec((1, C, tw),
                               lambda b, k: (b, 0, jnp.maximum(k - kt, 0))),
        scratch_shapes=[pltpu.VMEM((C, kt * tw), x.dtype),
                        pltpu.VMEM((C, 128), jnp.float32),
                        pltpu.VMEM((C, 1), jnp.float32)],
        compiler_params=pltpu.CompilerParams(
            dimension_semantics=("parallel", "arbitrary"),
            vmem_limit_bytes=_VMEM_LIMIT),
    )(x_flat, w1t, b1, w2, b2)
    return y.reshape(B, C, H, W)


def _pick_th(C, H, W, itemsize, tile_bytes=_TILE_BYTES):
    if W % 128 != 0 or H % 8 != 0:
        return None
    best = None
    for th in range(8, H + 1, 8):
        if H % th == 0 and C * th * W * itemsize <= tile_bytes:
            best = th
    return best if best is not None else 8


def kernel(x, conv1_w, conv1_b, conv2_w, conv2_b):
    B, C, H, W = x.shape
    cr = conv1_w.shape[0]

    w1t = conv1_w.astype(jnp.float32).T
    b1 = conv1_b.astype(jnp.float32).reshape(1, cr)
    w2 = conv2_w.astype(jnp.float32)
    b2 = conv2_b.astype(jnp.float32).reshape(C, 1)

    th = _pick_th(C, H, W, x.dtype.itemsize)
    if th is not None:
        if B % _NCORES == 0 and B >= 2 * _NCORES:
            slab_bytes = C * H * W * x.dtype.itemsize
            tile_budget = max((48 * 1024 * 1024 - slab_bytes) // 4,
                              _TILE_BYTES)
            th_pipe = _pick_th(C, H, W, x.dtype.itemsize, tile_budget)
            return _ca_layer_pipe(x, w1t, b1, w2, b2, th_pipe)
        return _ca_layer4d(x, w1t, b1, w2, b2, th)
    return _ca_layer_flat(x, w1t, b1, w2, b2)
